# baseline (speedup 1.0000x reference)
import functools

import jax
import jax.numpy as jnp
from jax import lax
from jax.experimental import pallas as pl
from jax.experimental.pallas import tpu as pltpu
from jax.experimental.pallas import tpu_sc as plsc

B, L, D = 4, 8192, 768
BUCKET = 8
PAD = BUCKET // 2
ROWS = 2 * B

PBLK = 256
QBLK = 1024
CBLK = 1024
LP = L + CBLK

NC, NS = 2, 16
NW = NC * NS
SPAN = ROWS * L // NW
CH = 128
NCHUNK = SPAN // CH

_mesh = plsc.VectorSubcoreMesh(core_axis_name="c", subcore_axis_name="s")


def _rank_body(kp_ref, kq_ref, out_ref):
    i = pl.program_id(1)
    q = pl.program_id(2)
    kp = kp_ref[0]
    kq = kq_ref[0]
    pstart = i * PBLK
    qstart = q * QBLK

    @pl.when(q == 0)
    def _():
        out_ref[0] = jnp.zeros((1, PBLK), jnp.int32)

    def accum(less):
        out_ref[0] += jnp.sum(less.astype(jnp.int32), axis=0, keepdims=True)

    @pl.when(qstart + QBLK <= pstart)
    def _():
        accum(kq <= kp)

    @pl.when(qstart >= pstart + PBLK)
    def _():
        accum(kq < kp)

    @pl.when((qstart + QBLK > pstart) & (qstart < pstart + PBLK))
    def _():
        ip = lax.broadcasted_iota(jnp.int32, (1, PBLK), 1) + pstart
        iq = lax.broadcasted_iota(jnp.int32, (QBLK, 1), 0) + qstart
        accum((kq < kp) | ((kq == kp) & (iq < ip)))


def _rank(keys8):
    return pl.pallas_call(
        _rank_body,
        grid=(ROWS, L // PBLK, L // QBLK),
        in_specs=[
            pl.BlockSpec((1, 1, PBLK), lambda r, i, q: (r, 0, i)),
            pl.BlockSpec((1, QBLK, 1), lambda r, i, q: (r, q, 0)),
        ],
        out_specs=pl.BlockSpec((1, 1, PBLK), lambda r, i, q: (r, 0, i)),
        out_shape=jax.ShapeDtypeStruct((ROWS, 1, L), jnp.int32),
    )(keys8.reshape(ROWS, 1, L), keys8.reshape(ROWS, L, 1)).reshape(ROWS, L)


@functools.partial(
    pl.kernel, mesh=_mesh,
    out_type=jax.ShapeDtypeStruct((ROWS * LP, D), jnp.float32),
    scratch_types=[
        pltpu.VMEM((CH,), jnp.int32),
        pltpu.VMEM((CH, D), jnp.float32),
        pltpu.SemaphoreType.DMA,
    ],
)
def _sc_scatter(x_hbm, rankp_hbm, out_hbm, idx_v, rows_v, sem):
    w = lax.axis_index("s") * NC + lax.axis_index("c")
    g = w // (NW // ROWS)
    q = w % (NW // ROWS)
    b = g % B
    pbase = q * SPAN

    def body(j, _):
        p0 = pbase + j * CH
        pltpu.sync_copy(rankp_hbm.at[pl.ds(g * L + p0, CH)], idx_v)
        pltpu.sync_copy(x_hbm.at[pl.ds(b * L + p0, CH), :], rows_v)
        pltpu.async_copy(rows_v, out_hbm.at[idx_v], sem).wait()
        return ()

    lax.fori_loop(0, NCHUNK, body, ())


@functools.partial(
    pl.kernel, mesh=_mesh,
    out_type=jax.ShapeDtypeStruct((ROWS * L, D), jnp.float32),
    scratch_types=[
        pltpu.VMEM((CH,), jnp.int32),
        pltpu.VMEM((CH, D), jnp.float32),
        pltpu.SemaphoreType.DMA,
    ],
)
def _sc_gather(h_hbm, rankg_hbm, out_hbm, idx_v, rows_v, sem):
    w = lax.axis_index("s") * NC + lax.axis_index("c")
    base = w * SPAN

    def body(j, _):
        p0 = base + j * CH
        pltpu.sync_copy(rankg_hbm.at[pl.ds(p0, CH)], idx_v)
        pltpu.async_copy(h_hbm.at[idx_v], rows_v, sem).wait()
        pltpu.sync_copy(rows_v, out_hbm.at[pl.ds(p0, CH), :])
        return ()

    lax.fori_loop(0, NCHUNK, body, ())


def _conv_body(a_ref, b_ref, w_ref, out_ref):
    i = pl.program_id(1)
    nb = pl.num_programs(1)
    xcat = jnp.concatenate([a_ref[0], b_ref[0]], axis=0)

    def compute(masked):
        rowid = lax.broadcasted_iota(jnp.int32, (CBLK, 1), 0) + i * CBLK
        acc = None
        for t in range(BUCKET):
            sl = xcat[t:t + CBLK]
            if masked:
                m = rowid + t
                sl = jnp.where((m >= PAD) & (m < L + PAD), sl, 0.0)
            term = sl * w_ref[0, t, :][None, :]
            acc = term if acc is None else acc + term
        return acc

    @pl.when((i == 0) | (i == nb - 1))
    def _():
        out_ref[0] = compute(True)

    @pl.when((i > 0) & (i < nb - 1))
    def _():
        out_ref[0] = compute(False)


def _conv(xpad, w28):
    return pl.pallas_call(
        _conv_body,
        grid=(ROWS, L // CBLK),
        in_specs=[
            pl.BlockSpec((1, CBLK, D), lambda r, i: (r, i, 0)),
            pl.BlockSpec((1, CBLK, D), lambda r, i: (r, i + 1, 0)),
            pl.BlockSpec((1, BUCKET, D), lambda r, i: (r // B, 0, 0)),
        ],
        out_specs=pl.BlockSpec((1, CBLK, D), lambda r, i: (r, i, 0)),
        out_shape=jax.ShapeDtypeStruct((ROWS, L, D), jnp.float32),
    )(xpad, xpad, w28)


def _finale_body(x_ref, g0_ref, g1_ref, w1t_ref, w2t_ref, ksum_ref, gamma_ref,
                 beta_ref, out_ref):
    x = x_ref[...]
    routed = (g0_ref[...] + g1_ref[...]) * 0.5
    logits = jnp.dot(x, w1t_ref[...], preferred_element_type=jnp.float32)
    logits += jnp.dot(routed, w2t_ref[...], preferred_element_type=jnp.float32)
    gate = jax.nn.sigmoid(logits + ksum_ref[...])
    y = x + gate * routed
    mean = jnp.mean(y, axis=-1, keepdims=True)
    yc = y - mean
    var = jnp.mean(yc * yc, axis=-1, keepdims=True)
    out_ref[...] = gamma_ref[...] * yc * lax.rsqrt(var + 1e-5) + beta_ref[...]


def _finale(x2d, g0, g1, w1t, w2t, ksum, gamma, beta):
    n = x2d.shape[0]
    blk = 1024
    return pl.pallas_call(
        _finale_body,
        grid=(n // blk,),
        in_specs=[
            pl.BlockSpec((blk, D), lambda i: (i, 0)),
            pl.BlockSpec((blk, D), lambda i: (i, 0)),
            pl.BlockSpec((blk, D), lambda i: (i, 0)),
            pl.BlockSpec((D, D), lambda i: (0, 0)),
            pl.BlockSpec((D, D), lambda i: (0, 0)),
            pl.BlockSpec((blk, 1), lambda i: (i, 0)),
            pl.BlockSpec((1, D), lambda i: (0, 0)),
            pl.BlockSpec((1, D), lambda i: (0, 0)),
        ],
        out_specs=pl.BlockSpec((blk, D), lambda i: (i, 0)),
        out_shape=jax.ShapeDtypeStruct((n, D), jnp.float32),
    )(x2d, g0, g1, w1t, w2t, ksum, gamma, beta)


def kernel(x, W_route, conv_w0, conv_w1, W_gate, ln_gamma, ln_beta):
    route_keys = x @ W_route.T
    keys8 = jnp.transpose(route_keys, (2, 0, 1)).reshape(ROWS, L)
    rank = _rank(keys8)

    goff = jnp.arange(ROWS, dtype=jnp.int32)[:, None]
    rankp = (rank + PAD + goff * LP).reshape(ROWS * L)
    rankg = (rank + goff * L).reshape(ROWS * L)

    x_flat = x.reshape(B * L, D)
    xpad = _sc_scatter(x_flat, rankp).reshape(ROWS, LP, D)

    w28 = jnp.stack([conv_w0[:, 0, :].T, conv_w1[:, 0, :].T])
    h = _conv(xpad, w28)

    g_flat = _sc_gather(h.reshape(ROWS * L, D), rankg)

    ksum = jnp.sum(route_keys, axis=-1).reshape(B * L, 1)
    out = _finale(
        x_flat,
        g_flat[:B * L],
        g_flat[B * L:],
        W_gate[:, :D].T,
        W_gate[:, D:].T,
        ksum,
        ln_gamma.reshape(1, D),
        ln_beta.reshape(1, D),
    )
    return out.reshape(B, L, D)


# speedup vs baseline: 1.7679x; 1.7679x over previous
import functools

import jax
import jax.numpy as jnp
from jax import lax
from jax.experimental import pallas as pl
from jax.experimental.pallas import tpu as pltpu
from jax.experimental.pallas import tpu_sc as plsc

B, L, D = 4, 8192, 768
BUCKET = 8
PAD = BUCKET // 2
G = B

PBLK = 256
CBLK = 512
LP = L + CBLK

NC, NS = 2, 16
NW = NC * NS
SPAN = G * L // NW
CH = 128
NCHUNK = SPAN // CH

_mesh = plsc.VectorSubcoreMesh(core_axis_name="c", subcore_axis_name="s")


def _rank_body(kp_ref, kq_ref, out_ref):
    i = pl.program_id(1)
    kp = kp_ref[0]
    ip = lax.broadcasted_iota(jnp.int32, (1, PBLK), 1) + i * PBLK
    kq = kq_ref[0]
    iq = lax.broadcasted_iota(jnp.int32, (L, 1), 0)
    less = (kq < kp) | ((kq == kp) & (iq < ip))
    cnt = jnp.sum(less.astype(jnp.float32), axis=0, keepdims=True)
    out_ref[0] = cnt.astype(jnp.int32)


def _rank(keys):
    return pl.pallas_call(
        _rank_body,
        grid=(G, L // PBLK),
        in_specs=[
            pl.BlockSpec((1, 1, PBLK), lambda r, i: (r, 0, i)),
            pl.BlockSpec((1, L, 1), lambda r, i: (r, 0, 0)),
        ],
        out_specs=pl.BlockSpec((1, 1, PBLK), lambda r, i: (r, 0, i)),
        out_shape=jax.ShapeDtypeStruct((G, 1, L), jnp.int32),
    )(keys.reshape(G, 1, L), keys.reshape(G, L, 1)).reshape(G, L)


@functools.partial(
    pl.kernel, mesh=_mesh,
    out_type=jax.ShapeDtypeStruct((G * LP, D), jnp.float32),
    scratch_types=[
        pltpu.VMEM((CH,), jnp.int32),
        pltpu.VMEM((CH, D), jnp.float32),
        pltpu.SemaphoreType.DMA,
    ],
)
def _sc_scatter(x_hbm, rankp_hbm, out_hbm, idx_v, rows_v, sem):
    w = lax.axis_index("s") * NC + lax.axis_index("c")
    base = w * SPAN

    def body(j, _):
        p0 = base + j * CH
        pltpu.sync_copy(rankp_hbm.at[pl.ds(p0, CH)], idx_v)
        pltpu.sync_copy(x_hbm.at[pl.ds(p0, CH), :], rows_v)
        pltpu.async_copy(rows_v, out_hbm.at[idx_v], sem).wait()
        return ()

    lax.fori_loop(0, NCHUNK, body, ())


@functools.partial(
    pl.kernel, mesh=_mesh,
    out_type=jax.ShapeDtypeStruct((G * L, D), jnp.float32),
    scratch_types=[
        pltpu.VMEM((CH,), jnp.int32),
        pltpu.VMEM((CH, D), jnp.float32),
        pltpu.SemaphoreType.DMA,
    ],
)
def _sc_gather(h_hbm, rankg_hbm, out_hbm, idx_v, rows_v, sem):
    w = lax.axis_index("s") * NC + lax.axis_index("c")
    base = w * SPAN

    def body(j, _):
        p0 = base + j * CH
        pltpu.sync_copy(rankg_hbm.at[pl.ds(p0, CH)], idx_v)
        pltpu.async_copy(h_hbm.at[idx_v], rows_v, sem).wait()
        pltpu.sync_copy(rows_v, out_hbm.at[pl.ds(p0, CH), :])
        return ()

    lax.fori_loop(0, NCHUNK, body, ())


def _conv_body(a_ref, b_ref, w_ref, out_ref):
    i = pl.program_id(1)
    xcat = jnp.concatenate([a_ref[0], b_ref[0]], axis=0)
    rowid = lax.broadcasted_iota(jnp.int32, (CBLK, 1), 0) + i * CBLK
    acc = None
    for t in range(BUCKET):
        m = rowid + t
        valid = (m >= PAD) & (m < L + PAD)
        sl = jnp.where(valid, xcat[t:t + CBLK], 0.0)
        term = sl * w_ref[0, t, :][None, :]
        acc = term if acc is None else acc + term
    out_ref[0] = acc


def _conv(xpad, w8):
    return pl.pallas_call(
        _conv_body,
        grid=(G, L // CBLK),
        in_specs=[
            pl.BlockSpec((1, CBLK, D), lambda r, i: (r, i, 0)),
            pl.BlockSpec((1, CBLK, D), lambda r, i: (r, i + 1, 0)),
            pl.BlockSpec((1, BUCKET, D), lambda r, i: (0, 0, 0)),
        ],
        out_specs=pl.BlockSpec((1, CBLK, D), lambda r, i: (r, i, 0)),
        out_shape=jax.ShapeDtypeStruct((G, L, D), jnp.float32),
    )(xpad, xpad, w8)


def _finale_body(x_ref, g0_ref, g1_ref, w1t_ref, w2t_ref, ksum_ref, gamma_ref,
                 beta_ref, out_ref):
    x = x_ref[...]
    routed = (g0_ref[...] + g1_ref[...]) * 0.5
    logits = jnp.dot(x.astype(jnp.bfloat16), w1t_ref[...],
                     preferred_element_type=jnp.float32)
    logits += jnp.dot(routed.astype(jnp.bfloat16), w2t_ref[...],
                      preferred_element_type=jnp.float32)
    gate = jax.nn.sigmoid(logits + ksum_ref[...])
    y = x + gate * routed
    mean = jnp.mean(y, axis=-1, keepdims=True)
    yc = y - mean
    var = jnp.mean(yc * yc, axis=-1, keepdims=True)
    out_ref[...] = gamma_ref[...] * yc * lax.rsqrt(var + 1e-5) + beta_ref[...]


def _finale(x2d, g0, g1, w1t, w2t, ksum, gamma, beta):
    n = x2d.shape[0]
    blk = 1024
    return pl.pallas_call(
        _finale_body,
        grid=(n // blk,),
        in_specs=[
            pl.BlockSpec((blk, D), lambda i: (i, 0)),
            pl.BlockSpec((blk, D), lambda i: (i, 0)),
            pl.BlockSpec((blk, D), lambda i: (i, 0)),
            pl.BlockSpec((D, D), lambda i: (0, 0)),
            pl.BlockSpec((D, D), lambda i: (0, 0)),
            pl.BlockSpec((blk, 1), lambda i: (i, 0)),
            pl.BlockSpec((1, D), lambda i: (0, 0)),
            pl.BlockSpec((1, D), lambda i: (0, 0)),
        ],
        out_specs=pl.BlockSpec((blk, D), lambda i: (i, 0)),
        out_shape=jax.ShapeDtypeStruct((n, D), jnp.float32),
    )(x2d, g0, g1, w1t, w2t, ksum, gamma, beta)


def kernel(x, W_route, conv_w0, conv_w1, W_gate, ln_gamma, ln_beta):
    route_keys = x @ W_route.T
    x_flat = x.reshape(B * L, D)
    goff = jnp.arange(G, dtype=jnp.int32)[:, None]

    keys = [route_keys[:, :, 0], route_keys[:, :, 1]]
    conv_ws = [conv_w0[:, 0, :].T[None], conv_w1[:, 0, :].T[None]]

    g_out = []
    for r in range(2):
        rank = _rank(keys[r])
        rankp = (rank + PAD + goff * LP).reshape(G * L)
        rankg = (rank + goff * L).reshape(G * L)
        xpad = _sc_scatter(x_flat, rankp).reshape(G, LP, D)
        h = _conv(xpad, conv_ws[r])
        g_out.append(_sc_gather(h.reshape(G * L, D), rankg))

    ksum = jnp.sum(route_keys, axis=-1).reshape(B * L, 1)
    out = _finale(
        x_flat,
        g_out[0],
        g_out[1],
        W_gate[:, :D].T.astype(jnp.bfloat16),
        W_gate[:, D:].T.astype(jnp.bfloat16),
        ksum,
        ln_gamma.reshape(1, D),
        ln_beta.reshape(1, D),
    )
    return out.reshape(B, L, D)
